# baseline (speedup 1.0000x reference)
import jax
import jax.numpy as jnp
from jax.experimental import pallas as pl
from jax.experimental.pallas import tpu as pltpu

B, N, F, E, A, H, DIAMETER = 32, 64, 128, 16, 32, 128, 3
BG = 4
JC = 16

_BIG = 1e30


def _round(h, eWm, Wi, Wj, Wuh, Wua, bm, bu):
    ai = jnp.dot(h, Wi, preferred_element_type=jnp.float32) + bm
    aj = jnp.dot(h, Wj, preferred_element_type=jnp.float32)
    ai4 = ai.reshape(BG, N, 1, H)
    aj3 = aj.reshape(BG, N, H)
    acc = None
    for jc in range(0, N, JC):
        aj_c = aj3[:, jc:jc + JC, :].reshape(BG, 1, JC, H)
        m_c = jax.nn.relu(ai4 + aj_c + eWm[:, :, jc:jc + JC, :])
        acc = m_c if acc is None else acc + m_c
    agg = jnp.sum(acc, axis=2)
    return jax.nn.relu(
        jnp.dot(h, Wuh, preferred_element_type=jnp.float32)
        + jnp.dot(agg.reshape(BG * N, H), Wua,
                  preferred_element_type=jnp.float32)
        + bu
    )


def _mpnn_both(h0, e2, Wm_p, bm_p, Wu_p, bu_p, Wm_v, bm_v, Wu_v, bu_v):
    WeC = jnp.concatenate(
        [jnp.concatenate([Wm_p[2 * F:], Wm_v[2 * F:]], axis=1),
         jnp.full((1, 2 * H), _BIG, dtype=jnp.float32)], axis=0)
    eWc = jnp.dot(e2, WeC, preferred_element_type=jnp.float32)
    eWm_p = eWc[:, :H].reshape(BG, N, N, H)
    eWm_v = eWc[:, H:].reshape(BG, N, N, H)
    hp, hv = h0, h0
    for _ in range(DIAMETER):
        hp = _round(hp, eWm_p, Wm_p[:F], Wm_p[F:2 * F],
                    Wu_p[:F], Wu_p[F:], bm_p, bu_p)
        hv = _round(hv, eWm_v, Wm_v[:F], Wm_v[F:2 * F],
                    Wu_v[:F], Wu_v[F:], bm_v, bu_v)
    return (jnp.sum(hp.reshape(BG, N, F), axis=1),
            jnp.sum(hv.reshape(BG, N, F), axis=1))


def _kernel(node_ref, e_ref, adj_ref,
            Wm_p_ref, bm_p_ref, Wu_p_ref, bu_p_ref, Wo_p_ref, bo_p_ref,
            Wm_v_ref, bm_v_ref, Wu_v_ref, bu_v_ref, Wo_v_ref, bo_v_ref,
            out_p, out_v):
    h0 = node_ref[...].reshape(BG * N, F)
    adjm1 = adj_ref[...].astype(jnp.float32).reshape(BG, N, 1, N) - 1.0
    e17 = jnp.concatenate([e_ref[...], adjm1], axis=2)
    e2 = jnp.swapaxes(e17, 2, 3).reshape(BG * N * N, E + 1)

    pid = pl.program_id(0)
    sel = (jax.lax.broadcasted_iota(jnp.int32, (BG, B), 1)
           == pid * BG + jax.lax.broadcasted_iota(jnp.int32, (BG, B), 0)
           ).astype(jnp.float32)

    pooled_p, pooled_v = _mpnn_both(
        h0, e2,
        Wm_p_ref[...], bm_p_ref[...], Wu_p_ref[...], bu_p_ref[...],
        Wm_v_ref[...], bm_v_ref[...], Wu_v_ref[...], bu_v_ref[...])
    rows_p = jnp.dot(pooled_p, Wo_p_ref[...],
                     preferred_element_type=jnp.float32)[:, :A] + bo_p_ref[...]

    @pl.when(pid % 2 == 0)
    def _():
        out_p[0:BG, :] = rows_p

    @pl.when(pid % 2 == 1)
    def _():
        out_p[BG:2 * BG, :] = rows_p

    vals = jnp.dot(pooled_v, Wo_v_ref[...],
                   preferred_element_type=jnp.float32)[:, :1] + bo_v_ref[...]
    contrib_v = jax.lax.dot_general(
        vals, sel, (((0,), (0,)), ((), ())),
        preferred_element_type=jnp.float32).reshape(1, 1, B)
    out_v[...] = jnp.where(pid == 0, contrib_v, out_v[...] + contrib_v)


@jax.jit
def kernel(node_feature_mat, edge_feature_mat, adj_max,
           Wm_p, bm_p, Wu_p, bu_p, Wo_p, bo_p,
           Wm_v, bm_v, Wu_v, bu_v, Wo_v, bo_v):
    full = lambda *s: pl.BlockSpec(s, lambda i: (0,) * len(s))
    grid = B // BG

    Wo_p_pad = jnp.pad(Wo_p, ((0, 0), (0, F - A)))
    Wo_v_pad = jnp.pad(Wo_v, ((0, 0), (0, F - 1)))

    out_p, out_v = pl.pallas_call(
        _kernel,
        grid=(grid,),
        in_specs=[
            pl.BlockSpec((BG, N, F), lambda i: (i, 0, 0)),
            pl.BlockSpec((BG, N, E, N), lambda i: (i, 0, 0, 0)),
            pl.BlockSpec((BG, N, N), lambda i: (i, 0, 0)),
            full(2 * F + E, H), full(H), full(F + H, F), full(F),
            full(F, F), full(A),
            full(2 * F + E, H), full(H), full(F + H, F), full(F),
            full(F, F), full(1),
        ],
        out_specs=[
            pl.BlockSpec((2 * BG, A), lambda i: (i // 2, 0)),
            pl.BlockSpec((1, 1, B), lambda i: (0, 0, 0)),
        ],
        out_shape=[
            jax.ShapeDtypeStruct((B, A), jnp.float32),
            jax.ShapeDtypeStruct((1, 1, B), jnp.float32),
        ],
        compiler_params=pltpu.CompilerParams(
            dimension_semantics=("arbitrary",),
        ),
    )(node_feature_mat, jnp.transpose(edge_feature_mat, (0, 1, 3, 2)), adj_max,
      Wm_p, bm_p, Wu_p, bu_p, Wo_p_pad, bo_p,
      Wm_v, bm_v, Wu_v, bu_v, Wo_v_pad, bo_v)

    return out_p, out_v.reshape(-1)


# speedup vs baseline: 1.0187x; 1.0187x over previous
import jax
import jax.numpy as jnp
from jax.experimental import pallas as pl
from jax.experimental.pallas import tpu as pltpu

B, N, F, E, A, H, DIAMETER = 32, 64, 128, 16, 32, 128, 3
BG = 4
JC = 8

_BIG = 1e30


def _round(h, eWm, Wi, Wj, Wuh, Wua, bm, bu):
    ai = jnp.dot(h, Wi, preferred_element_type=jnp.float32) + bm
    aj = jnp.dot(h, Wj, preferred_element_type=jnp.float32)
    ai4 = ai.reshape(BG, N, 1, H)
    aj3 = aj.reshape(BG, N, H)
    acc = None
    for jc in range(0, N, JC):
        aj_c = aj3[:, jc:jc + JC, :].reshape(BG, 1, JC, H)
        m_c = jax.nn.relu(ai4 + aj_c + eWm[:, :, jc:jc + JC, :])
        acc = m_c if acc is None else acc + m_c
    agg = jnp.sum(acc, axis=2)
    return jax.nn.relu(
        jnp.dot(h, Wuh, preferred_element_type=jnp.float32)
        + jnp.dot(agg.reshape(BG * N, H), Wua,
                  preferred_element_type=jnp.float32)
        + bu
    )


def _mpnn_both(h0, e2, Wm_p, bm_p, Wu_p, bu_p, Wm_v, bm_v, Wu_v, bu_v):
    WeC = jnp.concatenate(
        [jnp.concatenate([Wm_p[2 * F:], Wm_v[2 * F:]], axis=1),
         jnp.full((1, 2 * H), _BIG, dtype=jnp.float32)], axis=0)
    eWc = jnp.dot(e2, WeC, preferred_element_type=jnp.float32)
    eWm_p = eWc[:, :H].reshape(BG, N, N, H)
    eWm_v = eWc[:, H:].reshape(BG, N, N, H)
    hp, hv = h0, h0
    for _ in range(DIAMETER):
        hp = _round(hp, eWm_p, Wm_p[:F], Wm_p[F:2 * F],
                    Wu_p[:F], Wu_p[F:], bm_p, bu_p)
        hv = _round(hv, eWm_v, Wm_v[:F], Wm_v[F:2 * F],
                    Wu_v[:F], Wu_v[F:], bm_v, bu_v)
    return (jnp.sum(hp.reshape(BG, N, F), axis=1),
            jnp.sum(hv.reshape(BG, N, F), axis=1))


def _kernel(node_ref, e_ref, adj_ref,
            Wm_p_ref, bm_p_ref, Wu_p_ref, bu_p_ref, Wo_p_ref, bo_p_ref,
            Wm_v_ref, bm_v_ref, Wu_v_ref, bu_v_ref, Wo_v_ref, bo_v_ref,
            out_p, out_v):
    h0 = node_ref[...].reshape(BG * N, F)
    adjm1 = adj_ref[...].astype(jnp.float32).reshape(BG, N, 1, N) - 1.0
    e17 = jnp.concatenate([e_ref[...], adjm1], axis=2)
    e2 = jnp.swapaxes(e17, 2, 3).reshape(BG * N * N, E + 1)

    pid = pl.program_id(0)
    sel = (jax.lax.broadcasted_iota(jnp.int32, (BG, B), 1)
           == pid * BG + jax.lax.broadcasted_iota(jnp.int32, (BG, B), 0)
           ).astype(jnp.float32)

    pooled_p, pooled_v = _mpnn_both(
        h0, e2,
        Wm_p_ref[...], bm_p_ref[...], Wu_p_ref[...], bu_p_ref[...],
        Wm_v_ref[...], bm_v_ref[...], Wu_v_ref[...], bu_v_ref[...])
    rows_p = jnp.dot(pooled_p, Wo_p_ref[...],
                     preferred_element_type=jnp.float32)[:, :A] + bo_p_ref[...]
    contrib_p = jax.lax.dot_general(
        sel, rows_p, (((0,), (0,)), ((), ())),
        preferred_element_type=jnp.float32).reshape(1, B, A)
    out_p[...] = jnp.where(pid == 0, contrib_p, out_p[...] + contrib_p)

    vals = jnp.dot(pooled_v, Wo_v_ref[...],
                   preferred_element_type=jnp.float32)[:, :1] + bo_v_ref[...]
    contrib_v = jax.lax.dot_general(
        vals, sel, (((0,), (0,)), ((), ())),
        preferred_element_type=jnp.float32).reshape(1, 1, B)
    out_v[...] = jnp.where(pid == 0, contrib_v, out_v[...] + contrib_v)


@jax.jit
def kernel(node_feature_mat, edge_feature_mat, adj_max,
           Wm_p, bm_p, Wu_p, bu_p, Wo_p, bo_p,
           Wm_v, bm_v, Wu_v, bu_v, Wo_v, bo_v):
    full = lambda *s: pl.BlockSpec(s, lambda i: (0,) * len(s))
    grid = B // BG

    Wo_p_pad = jnp.pad(Wo_p, ((0, 0), (0, F - A)))
    Wo_v_pad = jnp.pad(Wo_v, ((0, 0), (0, F - 1)))

    out_p, out_v = pl.pallas_call(
        _kernel,
        grid=(grid,),
        in_specs=[
            pl.BlockSpec((BG, N, F), lambda i: (i, 0, 0)),
            pl.BlockSpec((BG, N, E, N), lambda i: (i, 0, 0, 0)),
            pl.BlockSpec((BG, N, N), lambda i: (i, 0, 0)),
            full(2 * F + E, H), full(H), full(F + H, F), full(F),
            full(F, F), full(A),
            full(2 * F + E, H), full(H), full(F + H, F), full(F),
            full(F, F), full(1),
        ],
        out_specs=[
            pl.BlockSpec((1, B, A), lambda i: (0, 0, 0)),
            pl.BlockSpec((1, 1, B), lambda i: (0, 0, 0)),
        ],
        out_shape=[
            jax.ShapeDtypeStruct((1, B, A), jnp.float32),
            jax.ShapeDtypeStruct((1, 1, B), jnp.float32),
        ],
        compiler_params=pltpu.CompilerParams(
            dimension_semantics=("arbitrary",),
        ),
    )(node_feature_mat, jnp.transpose(edge_feature_mat, (0, 1, 3, 2)), adj_max,
      Wm_p, bm_p, Wu_p, bu_p, Wo_p_pad, bo_p,
      Wm_v, bm_v, Wu_v, bu_v, Wo_v_pad, bo_v)

    return out_p.reshape(B, A), out_v.reshape(-1)
